# baseline (speedup 1.0000x reference)
import jax
import jax.numpy as jnp
from jax.experimental import pallas as pl
from jax.experimental.pallas import tpu as pltpu
from jax.experimental.pallas import tpu_sc as plsc

BATCH = 16384
DIM = 32
PACK = 128 // DIM
WIDE = 128
GATHER_WINDOW = 128
MLP_BLOCK = 2048


def _sc_gather(ue_wide, ie_wide, uid4, iid4):
    mesh = plsc.VectorSubcoreMesh(core_axis_name="core", subcore_axis_name="subcore")
    uid = uid4.reshape(1, BATCH)
    iid = iid4.reshape(1, BATCH)
    out_t = (
        jax.ShapeDtypeStruct((BATCH, WIDE), jnp.float32),
        jax.ShapeDtypeStruct((BATCH, WIDE), jnp.float32),
    )

    @pl.kernel(out_type=out_t, mesh=mesh)
    def gather_kernel(ue_hbm, ie_hbm, uid_hbm, iid_hbm, ue_out, ie_out):
        def body(uid_vmem, iid_vmem, ue_vmem, ie_vmem):
            pltpu.sync_copy(ue_hbm.at[uid_vmem.at[0]], ue_vmem)
            pltpu.sync_copy(ie_hbm.at[iid_vmem.at[0]], ie_vmem)

        pltpu.emit_pipeline(
            body,
            grid=(BATCH // GATHER_WINDOW,),
            in_specs=[
                pl.BlockSpec((1, GATHER_WINDOW), lambda i: (0, i)),
                pl.BlockSpec((1, GATHER_WINDOW), lambda i: (0, i)),
            ],
            out_specs=[
                pl.BlockSpec((GATHER_WINDOW, WIDE), lambda i: (i, 0)),
                pl.BlockSpec((GATHER_WINDOW, WIDE), lambda i: (i, 0)),
            ],
            core_axis_name=("core", "subcore"),
            dimension_semantics=(pltpu.PARALLEL,),
        )(uid_hbm, iid_hbm, ue_out, ie_out)

    return gather_kernel(ue_wide, ie_wide, uid, iid)


def _select_quadrant(wide, q):
    out = jnp.zeros((wide.shape[0], DIM), jnp.float32)
    for k in range(PACK):
        mask = (q == k).astype(jnp.float32)
        out = out + mask * wide[:, k * DIM:(k + 1) * DIM]
    return out


def _mlp_body(wu_ref, wi_ref, qu_ref, qi_ref, w0a_ref, w0b_ref, b0_ref,
              w1_ref, b1_ref, w2_ref, b2_ref, wo_ref, bo_ref, o_ref):
    ue = _select_quadrant(wu_ref[...], qu_ref[...])
    ie = _select_quadrant(wi_ref[...], qi_ref[...])
    x = ue @ w0a_ref[...] + ie @ w0b_ref[...] + b0_ref[...]
    x = jnp.maximum(x, 0.0)
    x = jnp.maximum(x @ w1_ref[...] + b1_ref[...], 0.0)
    x = jnp.maximum(x @ w2_ref[...] + b2_ref[...], 0.0)
    o_ref[...] = jax.nn.sigmoid(x @ wo_ref[...] + bo_ref[...])


def _tc_mlp(wu, wi, qu, qi, W0, b0, W1, b1, W2, b2, Wout, bout):
    w0a = W0[:DIM]
    w0b = W0[DIM:]
    full = lambda shape: pl.BlockSpec(shape, lambda i: (0, 0))
    grid = (BATCH // MLP_BLOCK,)
    return pl.pallas_call(
        _mlp_body,
        grid=grid,
        in_specs=[
            pl.BlockSpec((MLP_BLOCK, WIDE), lambda i: (i, 0)),
            pl.BlockSpec((MLP_BLOCK, WIDE), lambda i: (i, 0)),
            pl.BlockSpec((MLP_BLOCK, 1), lambda i: (i, 0)),
            pl.BlockSpec((MLP_BLOCK, 1), lambda i: (i, 0)),
            full(w0a.shape),
            full(w0b.shape),
            full((1, b0.shape[0])),
            full(W1.shape),
            full((1, b1.shape[0])),
            full(W2.shape),
            full((1, b2.shape[0])),
            full(Wout.shape),
            full((1, bout.shape[0])),
        ],
        out_specs=pl.BlockSpec((MLP_BLOCK, 1), lambda i: (i, 0)),
        out_shape=jax.ShapeDtypeStruct((BATCH, 1), jnp.float32),
    )(wu, wi, qu, qi, w0a, w0b, b0.reshape(1, -1), W1, b1.reshape(1, -1),
      W2, b2.reshape(1, -1), Wout, bout.reshape(1, -1))


def kernel(user_id, item_id, user_emb, item_emb, W0, b0, W1, b1, W2, b2, Wout, bout):
    user_id = user_id.astype(jnp.int32)
    item_id = item_id.astype(jnp.int32)
    ue_wide = user_emb.reshape(user_emb.shape[0] // PACK, WIDE)
    ie_wide = item_emb.reshape(item_emb.shape[0] // PACK, WIDE)
    wu, wi = _sc_gather(ue_wide, ie_wide, user_id // PACK, item_id // PACK)
    qu = (user_id % PACK).reshape(BATCH, 1)
    qi = (item_id % PACK).reshape(BATCH, 1)
    return _tc_mlp(wu, wi, qu, qi, W0, b0, W1, b1, W2, b2, Wout, bout)
